# Initial kernel scaffold; baseline (speedup 1.0000x reference)
import jax
import jax.numpy as jnp
from jax.experimental import pallas as pl


def kernel(input, i_type, weight, bias):
    raise NotImplementedError("write your pallas kernel here")


# speedup vs baseline: 1.4051x; 1.4051x over previous
import functools

import jax
import jax.numpy as jnp
from jax import lax
from jax.experimental import pallas as pl
from jax.experimental.pallas import tpu as pltpu

_M_BLK = 256


def _grouped_mm_body(g_ref, nreal_ref, x_ref, w_ref, b_ref, o_ref):
    i = pl.program_id(0)

    @pl.when(i < nreal_ref[0])
    def _():
        acc = lax.dot_general(
            x_ref[...], w_ref[0],
            (((1,), (1,)), ((), ())),
            preferred_element_type=jnp.float32,
        )
        o_ref[...] = acc + b_ref[0]


def kernel(input, i_type, weight, bias):
    n_tokens, d_in = input.shape
    n_types, d_out, _ = weight.shape
    m = _M_BLK
    nb = n_tokens // m + n_types
    nbm = nb * m

    counts = jnp.bincount(i_type, length=n_types)
    padded = ((counts + m - 1) // m) * m
    pstart = jnp.concatenate([jnp.zeros((1,), jnp.int32),
                              jnp.cumsum(padded).astype(jnp.int32)])
    start = jnp.concatenate([jnp.zeros((1,), jnp.int32),
                             jnp.cumsum(counts).astype(jnp.int32)])
    total_padded = pstart[n_types]
    nreal = (total_padded // m).astype(jnp.int32)

    order = jnp.argsort(i_type)
    t_sorted = i_type[order]
    ppos = pstart[t_sorted] + (jnp.arange(n_tokens, dtype=jnp.int32)
                               - start[t_sorted])
    gather_idx = jnp.zeros((nbm,), jnp.int32).at[ppos].set(order)
    dest_idx = jnp.zeros((n_tokens,), jnp.int32).at[order].set(ppos)

    blk = jnp.arange(nb, dtype=jnp.int32)
    block_g = jnp.searchsorted(pstart[1:], blk * m, side="right").astype(jnp.int32)
    last_g = block_g[jnp.maximum(nreal - 1, 0)]
    block_g = jnp.where(blk < nreal, jnp.minimum(block_g, n_types - 1), last_g)

    x_sorted = jnp.take(input, gather_idx, axis=0)

    grid_spec = pltpu.PrefetchScalarGridSpec(
        num_scalar_prefetch=2,
        grid=(nb,),
        in_specs=[
            pl.BlockSpec((m, d_in), lambda i, g, nr: (i, 0)),
            pl.BlockSpec((1, d_out, d_in), lambda i, g, nr: (g[i], 0, 0)),
            pl.BlockSpec((1, 1, d_out), lambda i, g, nr: (g[i], 0, 0)),
        ],
        out_specs=pl.BlockSpec((m, d_out), lambda i, g, nr: (i, 0)),
    )
    out_sorted = pl.pallas_call(
        _grouped_mm_body,
        grid_spec=grid_spec,
        out_shape=jax.ShapeDtypeStruct((nbm, d_out), jnp.float32),
    )(block_g, nreal.reshape(1), x_sorted, weight,
      bias.reshape(n_types, 1, d_out))

    return jnp.take(out_sorted, dest_idx, axis=0)


# speedup vs baseline: 1.6736x; 1.1911x over previous
import functools

import jax
import jax.numpy as jnp
from jax import lax
from jax.experimental import pallas as pl
from jax.experimental.pallas import tpu as pltpu
from jax.experimental.pallas import tpu_sc as plsc

_M_BLK = 256
_GATHER_CHUNK = 16


@functools.lru_cache(maxsize=None)
def _make_sc_gather(n_rows, n_out, d):
    info = plsc.get_sparse_core_info()
    nw = info.num_cores * info.num_subcores
    assert n_out % (nw * _GATHER_CHUNK) == 0
    b_per_w = n_out // nw
    n_chunks = b_per_w // _GATHER_CHUNK
    c = _GATHER_CHUNK
    mesh = plsc.VectorSubcoreMesh(core_axis_name="c", subcore_axis_name="s")

    @functools.partial(
        pl.kernel,
        out_type=jax.ShapeDtypeStruct((n_out, d), jnp.float32),
        mesh=mesh,
        scratch_types=[
            pltpu.VMEM((b_per_w,), jnp.int32),
            pltpu.VMEM((c, d), jnp.float32),
            pltpu.VMEM((c, d), jnp.float32),
            pltpu.SemaphoreType.DMA,
            pltpu.SemaphoreType.DMA,
            pltpu.SemaphoreType.DMA,
            pltpu.SemaphoreType.DMA,
        ],
    )
    def sc_gather(table_hbm, idx_hbm, out_hbm, idx_v, buf0, buf1,
                  g_sem0, g_sem1, s_sem0, s_sem1):
        wid = lax.axis_index("s") * info.num_cores + lax.axis_index("c")
        base = wid * b_per_w
        pltpu.sync_copy(idx_hbm.at[pl.ds(base, b_per_w)], idx_v)

        bufs = (buf0, buf1)
        g_sems = (g_sem0, g_sem1)
        s_sems = (s_sem0, s_sem1)

        def gather_start(k):
            b = k & 1
            return pltpu.make_async_copy(
                table_hbm.at[idx_v.at[pl.ds(k * c, c)]], bufs[b], g_sems[b])

        def store_start(k):
            b = k & 1
            return pltpu.make_async_copy(
                bufs[b], out_hbm.at[pl.ds(base + k * c, c)], s_sems[b])

        gather_start(0).start()
        for k in range(n_chunks):
            if k + 1 < n_chunks:
                if k >= 1:
                    store_start(k - 1).wait()
                gather_start(k + 1).start()
            gather_start(k).wait()
            store_start(k).start()
        if n_chunks >= 2:
            store_start(n_chunks - 2).wait()
        store_start(n_chunks - 1).wait()

    return sc_gather


def _grouped_mm_body(g_ref, nreal_ref, x_ref, w_ref, b_ref, o_ref):
    i = pl.program_id(0)

    @pl.when(i < nreal_ref[0])
    def _():
        acc = lax.dot_general(
            x_ref[...], w_ref[0],
            (((1,), (1,)), ((), ())),
            preferred_element_type=jnp.float32,
        )
        o_ref[...] = acc + b_ref[0]


def kernel(input, i_type, weight, bias):
    n_tokens, d_in = input.shape
    n_types, d_out, _ = weight.shape
    m = _M_BLK
    nb = n_tokens // m + n_types
    nbm = nb * m

    counts = jnp.bincount(i_type, length=n_types)
    padded = ((counts + m - 1) // m) * m
    pstart = jnp.concatenate([jnp.zeros((1,), jnp.int32),
                              jnp.cumsum(padded).astype(jnp.int32)])
    start = jnp.concatenate([jnp.zeros((1,), jnp.int32),
                             jnp.cumsum(counts).astype(jnp.int32)])
    total_padded = pstart[n_types]
    nreal = (total_padded // m).astype(jnp.int32)

    order = jnp.argsort(i_type)
    t_sorted = i_type[order]
    ppos = pstart[t_sorted] + (jnp.arange(n_tokens, dtype=jnp.int32)
                               - start[t_sorted])
    gather_idx = jnp.zeros((nbm,), jnp.int32).at[ppos].set(order)
    dest_idx = jnp.zeros((n_tokens,), jnp.int32).at[order].set(ppos)

    blk = jnp.arange(nb, dtype=jnp.int32)
    block_g = jnp.searchsorted(pstart[1:], blk * m, side="right").astype(jnp.int32)
    last_g = block_g[jnp.maximum(nreal - 1, 0)]
    block_g = jnp.where(blk < nreal, jnp.minimum(block_g, n_types - 1), last_g)

    x_sorted = _make_sc_gather(n_tokens, nbm, d_in)(input, gather_idx)

    grid_spec = pltpu.PrefetchScalarGridSpec(
        num_scalar_prefetch=2,
        grid=(nb,),
        in_specs=[
            pl.BlockSpec((m, d_in), lambda i, g, nr: (i, 0)),
            pl.BlockSpec((1, d_out, d_in), lambda i, g, nr: (g[i], 0, 0)),
            pl.BlockSpec((1, 1, d_out), lambda i, g, nr: (g[i], 0, 0)),
        ],
        out_specs=pl.BlockSpec((m, d_out), lambda i, g, nr: (i, 0)),
    )
    out_sorted = pl.pallas_call(
        _grouped_mm_body,
        grid_spec=grid_spec,
        out_shape=jax.ShapeDtypeStruct((nbm, d_out), jnp.float32),
    )(block_g, nreal.reshape(1), x_sorted, weight,
      bias.reshape(n_types, 1, d_out))

    return _make_sc_gather(nbm, n_tokens, d_out)(out_sorted, dest_idx)


# speedup vs baseline: 2.4383x; 1.4569x over previous
import functools

import jax
import jax.numpy as jnp
from jax import lax
from jax.experimental import pallas as pl
from jax.experimental.pallas import tpu as pltpu
from jax.experimental.pallas import tpu_sc as plsc

_M_BLK = 256
_N_CHUNKS = 4
_DMA_ROWS = 16


@functools.lru_cache(maxsize=None)
def _sc_info():
    info = plsc.get_sparse_core_info()
    return info.num_cores, info.num_subcores


@functools.lru_cache(maxsize=None)
def _make_sc_gather(n_out, d):
    nc, ns = _sc_info()
    nw = nc * ns
    c = _DMA_ROWS
    assert n_out % (nw * c) == 0
    b_per_w = n_out // nw
    n_chunks = b_per_w // c
    mesh = plsc.VectorSubcoreMesh(core_axis_name="c", subcore_axis_name="s")

    @functools.partial(
        pl.kernel,
        out_type=jax.ShapeDtypeStruct((n_out, d), jnp.float32),
        mesh=mesh,
        scratch_types=[
            pltpu.VMEM((b_per_w,), jnp.int32),
            pltpu.VMEM((c, d), jnp.float32),
            pltpu.VMEM((c, d), jnp.float32),
            pltpu.SemaphoreType.DMA,
            pltpu.SemaphoreType.DMA,
            pltpu.SemaphoreType.DMA,
            pltpu.SemaphoreType.DMA,
        ],
    )
    def sc_gather(table_hbm, idx_hbm, out_hbm, idx_v, buf0, buf1,
                  g_sem0, g_sem1, s_sem0, s_sem1):
        wid = lax.axis_index("s") * nc + lax.axis_index("c")
        base = wid * b_per_w
        pltpu.sync_copy(idx_hbm.at[pl.ds(base, b_per_w)], idx_v)

        bufs = (buf0, buf1)
        g_sems = (g_sem0, g_sem1)
        s_sems = (s_sem0, s_sem1)

        def gather_k(k):
            b = k & 1
            return pltpu.make_async_copy(
                table_hbm.at[idx_v.at[pl.ds(k * c, c)]], bufs[b], g_sems[b])

        def store_k(k):
            b = k & 1
            return pltpu.make_async_copy(
                bufs[b], out_hbm.at[pl.ds(base + k * c, c)], s_sems[b])

        gather_k(0).start()
        for k in range(n_chunks):
            if k + 1 < n_chunks:
                if k >= 1:
                    store_k(k - 1).wait()
                gather_k(k + 1).start()
            gather_k(k).wait()
            store_k(k).start()
        if n_chunks >= 2:
            store_k(n_chunks - 2).wait()
        store_k(n_chunks - 1).wait()

    return sc_gather


@functools.lru_cache(maxsize=None)
def _make_sc_scatter(n_in, d):
    nc, ns = _sc_info()
    nw = nc * ns
    c = _DMA_ROWS
    assert n_in % (nw * c) == 0
    b_per_w = n_in // nw
    n_chunks = b_per_w // c
    mesh = plsc.VectorSubcoreMesh(core_axis_name="c", subcore_axis_name="s")

    @functools.partial(
        pl.kernel,
        mesh=mesh,
        scratch_types=[
            pltpu.VMEM((n_chunks, c), jnp.int32),
            pltpu.VMEM((c, d), jnp.float32),
            pltpu.VMEM((c, d), jnp.float32),
            pltpu.SemaphoreType.DMA,
            pltpu.SemaphoreType.DMA,
            pltpu.SemaphoreType.DMA,
            pltpu.SemaphoreType.DMA,
        ],
    )
    def sc_scatter(rows_hbm, idx_hbm, out_ref, idx_v, buf0, buf1,
                   l_sem0, l_sem1, s_sem0, s_sem1):
        wid = lax.axis_index("s") * nc + lax.axis_index("c")
        base = wid * b_per_w
        pltpu.sync_copy(idx_hbm.at[wid], idx_v)

        bufs = (buf0, buf1)
        l_sems = (l_sem0, l_sem1)
        s_sems = (s_sem0, s_sem1)

        def load_k(k):
            b = k & 1
            return pltpu.make_async_copy(
                rows_hbm.at[pl.ds(base + k * c, c)], bufs[b], l_sems[b])

        def scat_k(k):
            b = k & 1
            return pltpu.make_async_copy(
                bufs[b], out_ref.at[idx_v.at[k]], s_sems[b])

        load_k(0).start()
        for k in range(n_chunks):
            if k + 1 < n_chunks:
                if k >= 1:
                    scat_k(k - 1).wait()
                load_k(k + 1).start()
            load_k(k).wait()
            scat_k(k).start()
        if n_chunks >= 2:
            scat_k(n_chunks - 2).wait()
        scat_k(n_chunks - 1).wait()

    return sc_scatter


def _grouped_mm_body(g_ref, x_ref, w_ref, b_ref, o_ref):
    acc = lax.dot_general(
        x_ref[...], w_ref[0],
        (((1,), (1,)), ((), ())),
        preferred_element_type=jnp.float32,
    )
    o_ref[...] = acc + b_ref[0]


def kernel(input, i_type, weight, bias):
    n_tokens, d_in = input.shape
    n_types, d_out, _ = weight.shape
    m = _M_BLK
    nb = n_tokens // m + n_types
    nbm = nb * m
    nw = _sc_info()[0] * _sc_info()[1]

    tk = jnp.arange(n_types, dtype=jnp.int32)
    counts = jnp.sum(i_type[:, None] == tk[None, :], axis=0, dtype=jnp.int32)
    padded = ((counts + m - 1) // m) * m
    pstart = jnp.concatenate([jnp.zeros((1,), jnp.int32),
                              jnp.cumsum(padded).astype(jnp.int32)])
    start = jnp.concatenate([jnp.zeros((1,), jnp.int32),
                             jnp.cumsum(counts).astype(jnp.int32)])
    t_last = jnp.max(tk * (counts > 0)).astype(jnp.int32)

    order = jnp.argsort(i_type).astype(jnp.int32)

    pos = jnp.arange(nbm, dtype=jnp.int32)
    t_raw = jnp.sum(pstart[1:][None, :] <= pos[:, None], axis=1,
                    dtype=jnp.int32)
    t_eff = jnp.minimum(t_raw, t_last)
    sel = t_eff[:, None] == tk[None, :]
    pstart_t = jnp.sum(jnp.where(sel, pstart[:-1][None, :], 0), axis=1)
    start_t = jnp.sum(jnp.where(sel, start[:-1][None, :], 0), axis=1)
    counts_t = jnp.sum(jnp.where(sel, counts[None, :], 0), axis=1)
    o = (pos - pstart_t) % jnp.maximum(counts_t, 1)
    src = order[jnp.clip(start_t + o, 0, n_tokens - 1)]

    block_g = t_eff[::m]

    ch_rows = nbm // _N_CHUNKS
    ch_blocks = nb // _N_CHUNKS
    src_sc = src.reshape(_N_CHUNKS, nw, ch_rows // (nw * _DMA_ROWS), _DMA_ROWS)
    bias3 = bias.reshape(n_types, 1, d_out)

    grid_spec = pltpu.PrefetchScalarGridSpec(
        num_scalar_prefetch=1,
        grid=(ch_blocks,),
        in_specs=[
            pl.BlockSpec((m, d_in), lambda i, g: (i, 0)),
            pl.BlockSpec((1, d_out, d_in), lambda i, g: (g[i], 0, 0)),
            pl.BlockSpec((1, 1, d_out), lambda i, g: (g[i], 0, 0)),
        ],
        out_specs=pl.BlockSpec((m, d_out), lambda i, g: (i, 0)),
    )
    mm = pl.pallas_call(
        _grouped_mm_body,
        grid_spec=grid_spec,
        out_shape=jax.ShapeDtypeStruct((ch_rows, d_out), jnp.float32),
    )
    gather_k = _make_sc_gather(ch_rows, d_in)
    scatter_k = _make_sc_scatter(ch_rows, d_out)

    out_ref = jax.new_ref(jnp.zeros((n_tokens, d_out), jnp.float32))
    for c in range(_N_CHUNKS):
        xs = gather_k(input, lax.dynamic_slice_in_dim(src, c * ch_rows, ch_rows))
        ys = mm(lax.dynamic_slice_in_dim(block_g, c * ch_blocks, ch_blocks),
                xs, weight, bias3)
        scatter_k(ys, src_sc[c], out_ref)
    return out_ref[...]


# speedup vs baseline: 2.5197x; 1.0334x over previous
import functools

import jax
import jax.numpy as jnp
from jax import lax
from jax.experimental import pallas as pl
from jax.experimental.pallas import tpu as pltpu
from jax.experimental.pallas import tpu_sc as plsc

_M_BLK = 256
_N_CHUNKS = 2
_DMA_ROWS = 16


@functools.lru_cache(maxsize=None)
def _sc_info():
    info = plsc.get_sparse_core_info()
    return info.num_cores, info.num_subcores


@functools.lru_cache(maxsize=None)
def _make_sc_gather(n_out, d):
    nc, ns = _sc_info()
    nw = nc * ns
    c = _DMA_ROWS
    assert n_out % (nw * c) == 0
    b_per_w = n_out // nw
    n_chunks = b_per_w // c
    mesh = plsc.VectorSubcoreMesh(core_axis_name="c", subcore_axis_name="s")

    @functools.partial(
        pl.kernel,
        out_type=jax.ShapeDtypeStruct((n_out, d), jnp.float32),
        mesh=mesh,
        scratch_types=[
            pltpu.VMEM((b_per_w,), jnp.int32),
            pltpu.VMEM((c, d), jnp.float32),
            pltpu.VMEM((c, d), jnp.float32),
            pltpu.SemaphoreType.DMA,
            pltpu.SemaphoreType.DMA,
            pltpu.SemaphoreType.DMA,
            pltpu.SemaphoreType.DMA,
        ],
    )
    def sc_gather(table_hbm, idx_hbm, out_hbm, idx_v, buf0, buf1,
                  g_sem0, g_sem1, s_sem0, s_sem1):
        wid = lax.axis_index("s") * nc + lax.axis_index("c")
        base = wid * b_per_w
        pltpu.sync_copy(idx_hbm.at[pl.ds(base, b_per_w)], idx_v)

        bufs = (buf0, buf1)
        g_sems = (g_sem0, g_sem1)
        s_sems = (s_sem0, s_sem1)

        def gather_k(k):
            b = k & 1
            return pltpu.make_async_copy(
                table_hbm.at[idx_v.at[pl.ds(k * c, c)]], bufs[b], g_sems[b])

        def store_k(k):
            b = k & 1
            return pltpu.make_async_copy(
                bufs[b], out_hbm.at[pl.ds(base + k * c, c)], s_sems[b])

        gather_k(0).start()
        for k in range(n_chunks):
            if k + 1 < n_chunks:
                if k >= 1:
                    store_k(k - 1).wait()
                gather_k(k + 1).start()
            gather_k(k).wait()
            store_k(k).start()
        if n_chunks >= 2:
            store_k(n_chunks - 2).wait()
        store_k(n_chunks - 1).wait()

    return sc_gather


@functools.lru_cache(maxsize=None)
def _make_sc_scatter(n_in, d):
    nc, ns = _sc_info()
    nw = nc * ns
    c = _DMA_ROWS
    assert n_in % (nw * c) == 0
    b_per_w = n_in // nw
    n_chunks = b_per_w // c
    mesh = plsc.VectorSubcoreMesh(core_axis_name="c", subcore_axis_name="s")

    @functools.partial(
        pl.kernel,
        mesh=mesh,
        scratch_types=[
            pltpu.VMEM((n_chunks, c), jnp.int32),
            pltpu.VMEM((c, d), jnp.float32),
            pltpu.VMEM((c, d), jnp.float32),
            pltpu.SemaphoreType.DMA,
            pltpu.SemaphoreType.DMA,
            pltpu.SemaphoreType.DMA,
            pltpu.SemaphoreType.DMA,
        ],
    )
    def sc_scatter(rows_hbm, idx_hbm, out_ref, idx_v, buf0, buf1,
                   l_sem0, l_sem1, s_sem0, s_sem1):
        wid = lax.axis_index("s") * nc + lax.axis_index("c")
        base = wid * b_per_w
        pltpu.sync_copy(idx_hbm.at[wid], idx_v)

        bufs = (buf0, buf1)
        l_sems = (l_sem0, l_sem1)
        s_sems = (s_sem0, s_sem1)

        def load_k(k):
            b = k & 1
            return pltpu.make_async_copy(
                rows_hbm.at[pl.ds(base + k * c, c)], bufs[b], l_sems[b])

        def scat_k(k):
            b = k & 1
            return pltpu.make_async_copy(
                bufs[b], out_ref.at[idx_v.at[k]], s_sems[b])

        load_k(0).start()
        for k in range(n_chunks):
            if k + 1 < n_chunks:
                if k >= 1:
                    scat_k(k - 1).wait()
                load_k(k + 1).start()
            load_k(k).wait()
            scat_k(k).start()
        if n_chunks >= 2:
            scat_k(n_chunks - 2).wait()
        scat_k(n_chunks - 1).wait()

    return sc_scatter


def _grouped_mm_body(g_ref, x_ref, w_ref, b_ref, o_ref):
    acc = lax.dot_general(
        x_ref[...], w_ref[0],
        (((1,), (1,)), ((), ())),
        preferred_element_type=jnp.float32,
    )
    o_ref[...] = acc + b_ref[0]


def kernel(input, i_type, weight, bias):
    n_tokens, d_in = input.shape
    n_types, d_out, _ = weight.shape
    m = _M_BLK
    nb = n_tokens // m + n_types
    nbm = nb * m
    nw = _sc_info()[0] * _sc_info()[1]

    tk = jnp.arange(n_types, dtype=jnp.int32)
    counts = jnp.sum(i_type[:, None] == tk[None, :], axis=0, dtype=jnp.int32)
    padded = ((counts + m - 1) // m) * m
    pstart = jnp.concatenate([jnp.zeros((1,), jnp.int32),
                              jnp.cumsum(padded).astype(jnp.int32)])
    start = jnp.concatenate([jnp.zeros((1,), jnp.int32),
                             jnp.cumsum(counts).astype(jnp.int32)])
    t_last = jnp.max(tk * (counts > 0)).astype(jnp.int32)

    order = jnp.argsort(i_type).astype(jnp.int32)

    pos = jnp.arange(nbm, dtype=jnp.int32)
    t_raw = jnp.sum(pstart[1:][None, :] <= pos[:, None], axis=1,
                    dtype=jnp.int32)
    t_eff = jnp.minimum(t_raw, t_last)
    sel = t_eff[:, None] == tk[None, :]
    pstart_t = jnp.sum(jnp.where(sel, pstart[:-1][None, :], 0), axis=1)
    start_t = jnp.sum(jnp.where(sel, start[:-1][None, :], 0), axis=1)
    counts_t = jnp.sum(jnp.where(sel, counts[None, :], 0), axis=1)
    o = (pos - pstart_t) % jnp.maximum(counts_t, 1)
    src = order[jnp.clip(start_t + o, 0, n_tokens - 1)]

    block_g = t_eff[::m]

    ch_rows = nbm // _N_CHUNKS
    ch_blocks = nb // _N_CHUNKS
    src_sc = src.reshape(_N_CHUNKS, nw, ch_rows // (nw * _DMA_ROWS), _DMA_ROWS)
    bias3 = bias.reshape(n_types, 1, d_out)

    grid_spec = pltpu.PrefetchScalarGridSpec(
        num_scalar_prefetch=1,
        grid=(ch_blocks,),
        in_specs=[
            pl.BlockSpec((m, d_in), lambda i, g: (i, 0)),
            pl.BlockSpec((1, d_out, d_in), lambda i, g: (g[i], 0, 0)),
            pl.BlockSpec((1, 1, d_out), lambda i, g: (g[i], 0, 0)),
        ],
        out_specs=pl.BlockSpec((m, d_out), lambda i, g: (i, 0)),
    )
    mm = pl.pallas_call(
        _grouped_mm_body,
        grid_spec=grid_spec,
        out_shape=jax.ShapeDtypeStruct((ch_rows, d_out), jnp.float32),
    )
    gather_k = _make_sc_gather(ch_rows, d_in)
    scatter_k = _make_sc_scatter(ch_rows, d_out)

    out_ref = jax.new_ref(jnp.zeros((n_tokens, d_out), jnp.float32))
    for c in range(_N_CHUNKS):
        xs = gather_k(input, lax.dynamic_slice_in_dim(src, c * ch_rows, ch_rows))
        ys = mm(lax.dynamic_slice_in_dim(block_g, c * ch_blocks, ch_blocks),
                xs, weight, bias3)
        scatter_k(ys, src_sc[c], out_ref)
    return out_ref[...]
